# Initial kernel scaffold; baseline (speedup 1.0000x reference)
import jax
import jax.numpy as jnp
from jax.experimental import pallas as pl


def kernel(z, pos, batch, emb, Wp, W1, b1, W2, b2):
    raise NotImplementedError("write your pallas kernel here")


# speedup vs baseline: 4.7518x; 4.7518x over previous
import jax
import jax.numpy as jnp
from jax.experimental import pallas as pl

N = 16384
B = 16
D = 256
H = 128
ZPAD = 128
BLK = 2048


def _fused_kernel(z_ref, pos_ref, batch_ref, emb_ref, wp_ref, w1_ref, b1_ref,
                  w2_ref, b2_ref, out_ref):
    i = pl.program_id(0)

    zb = z_ref[0, 0, :]
    oh = (zb[:, None] == jax.lax.broadcasted_iota(jnp.int32, (BLK, ZPAD), 1))
    xg = jnp.dot(oh.astype(jnp.float32), emb_ref[...],
                 preferred_element_type=jnp.float32)
    x = xg + jnp.dot(pos_ref[...], wp_ref[...],
                     preferred_element_type=jnp.float32)

    h = jnp.dot(x, w1_ref[...], preferred_element_type=jnp.float32)
    h = h + b1_ref[0, :][None, :]
    h = h * jax.nn.sigmoid(h)

    xa = jnp.dot(h, w2_ref[...], preferred_element_type=jnp.float32)
    xa = xa + b2_ref[0, 0]

    bb = batch_ref[0, 0, :]
    seg = (bb[:, None] == jax.lax.broadcasted_iota(jnp.int32, (BLK, B), 1))
    partial = jnp.sum(seg.astype(jnp.float32) * xa, axis=0)[:, None]

    @pl.when(i == 0)
    def _():
        out_ref[...] = jnp.zeros_like(out_ref)

    out_ref[...] += partial


@jax.jit
def kernel(z, pos, batch, emb, Wp, W1, b1, W2, b2):
    grid = N // BLK
    z3 = z.astype(jnp.int32).reshape(grid, 1, BLK)
    batch3 = batch.astype(jnp.int32).reshape(grid, 1, BLK)
    emb_pad = jnp.zeros((ZPAD, D), jnp.float32).at[:emb.shape[0]].set(emb)
    b1r = b1.reshape(1, H)
    b2r = b2.reshape(1, 1)

    out = pl.pallas_call(
        _fused_kernel,
        grid=(grid,),
        in_specs=[
            pl.BlockSpec((1, 1, BLK), lambda i: (i, 0, 0)),
            pl.BlockSpec((BLK, 3), lambda i: (i, 0)),
            pl.BlockSpec((1, 1, BLK), lambda i: (i, 0, 0)),
            pl.BlockSpec((ZPAD, D), lambda i: (0, 0)),
            pl.BlockSpec((3, D), lambda i: (0, 0)),
            pl.BlockSpec((D, H), lambda i: (0, 0)),
            pl.BlockSpec((1, H), lambda i: (0, 0)),
            pl.BlockSpec((H, 1), lambda i: (0, 0)),
            pl.BlockSpec((1, 1), lambda i: (0, 0)),
        ],
        out_specs=pl.BlockSpec((B, 1), lambda i: (0, 0)),
        out_shape=jax.ShapeDtypeStruct((B, 1), jnp.float32),
    )(z3, pos, batch3, emb_pad, Wp, W1, b1r, W2, b2r)
    return out


# speedup vs baseline: 5.6323x; 1.1853x over previous
import jax
import jax.numpy as jnp
from jax.experimental import pallas as pl
from jax.experimental.pallas import tpu as pltpu

N = 16384
B = 16
D = 256
H = 128
ZPAD = 128
BLK = 2048


def _fused_kernel(z_ref, pos_ref, batch_ref, emb_ref, wp_ref, w1_ref, b1_ref,
                  w2_ref, b2_ref, out_ref, embw1_ref, wpw1_ref):
    i = pl.program_id(0)

    @pl.when(i == 0)
    def _():
        embw1_ref[...] = jnp.dot(emb_ref[...], w1_ref[...],
                                 preferred_element_type=jnp.float32)
        wpw1_ref[...] = jnp.dot(wp_ref[...], w1_ref[...],
                                preferred_element_type=jnp.float32)
        out_ref[...] = jnp.zeros_like(out_ref)

    zb = z_ref[0, 0, :]
    oh = (zb[:, None] == jax.lax.broadcasted_iota(jnp.int32, (BLK, ZPAD), 1))
    hx = jnp.dot(oh.astype(jnp.float32), embw1_ref[...],
                 preferred_element_type=jnp.float32)
    hx = hx + jnp.dot(pos_ref[...], wpw1_ref[...],
                      preferred_element_type=jnp.float32)
    hx = hx + b1_ref[0, :][None, :]
    h = hx * jax.nn.sigmoid(hx)

    xa = jnp.dot(h, w2_ref[...], preferred_element_type=jnp.float32)
    xa = xa + b2_ref[0, 0]

    bb = batch_ref[0, 0, :]
    seg = (bb[:, None] == jax.lax.broadcasted_iota(jnp.int32, (BLK, B), 1))
    partial = jnp.sum(seg.astype(jnp.float32) * xa, axis=0)[:, None]

    out_ref[...] += partial


@jax.jit
def kernel(z, pos, batch, emb, Wp, W1, b1, W2, b2):
    grid = N // BLK
    z3 = z.astype(jnp.int32).reshape(grid, 1, BLK)
    batch3 = batch.astype(jnp.int32).reshape(grid, 1, BLK)
    emb_pad = jnp.zeros((ZPAD, D), jnp.float32).at[:emb.shape[0]].set(emb)
    b1r = b1.reshape(1, H)
    b2r = b2.reshape(1, 1)

    out = pl.pallas_call(
        _fused_kernel,
        grid=(grid,),
        in_specs=[
            pl.BlockSpec((1, 1, BLK), lambda i: (i, 0, 0)),
            pl.BlockSpec((BLK, 3), lambda i: (i, 0)),
            pl.BlockSpec((1, 1, BLK), lambda i: (i, 0, 0)),
            pl.BlockSpec((ZPAD, D), lambda i: (0, 0)),
            pl.BlockSpec((3, D), lambda i: (0, 0)),
            pl.BlockSpec((D, H), lambda i: (0, 0)),
            pl.BlockSpec((1, H), lambda i: (0, 0)),
            pl.BlockSpec((H, 1), lambda i: (0, 0)),
            pl.BlockSpec((1, 1), lambda i: (0, 0)),
        ],
        out_specs=pl.BlockSpec((B, 1), lambda i: (0, 0)),
        out_shape=jax.ShapeDtypeStruct((B, 1), jnp.float32),
        scratch_shapes=[
            pltpu.VMEM((ZPAD, H), jnp.float32),
            pltpu.VMEM((3, H), jnp.float32),
        ],
    )(z3, pos, batch3, emb_pad, Wp, W1, b1r, W2, b2r)
    return out


# speedup vs baseline: 7.0665x; 1.2546x over previous
import jax
import jax.numpy as jnp
from jax import lax
from jax.experimental import pallas as pl
from jax.experimental.pallas import tpu as pltpu

N = 16384
B = 16
D = 256
H = 128
ZMAXK = 100
ZPAD = 128
BLK = 4096


def _fused_kernel(z_ref, pos_ref, batch_ref, emb_ref, wp_ref, w1_ref, b1_ref,
                  w2_ref, b2_ref, out_ref, embw1t_ref, wpw1t_ref):
    i = pl.program_id(0)

    @pl.when(i == 0)
    def _():
        embw1t_ref[...] = jnp.zeros_like(embw1t_ref)
        embw1t_ref[:, 0:ZMAXK] = lax.dot_general(
            w1_ref[...], emb_ref[...], (((0,), (1,)), ((), ())),
            preferred_element_type=jnp.float32)
        wpw1t_ref[...] = lax.dot_general(
            w1_ref[...], wp_ref[...], (((0,), (1,)), ((), ())),
            preferred_element_type=jnp.float32)
        out_ref[...] = jnp.zeros_like(out_ref)

    zb = z_ref[0, :, :]
    oht = (jax.lax.broadcasted_iota(jnp.int32, (ZPAD, BLK), 0) == zb)
    hxt = jnp.dot(embw1t_ref[...], oht.astype(jnp.float32),
                  preferred_element_type=jnp.float32)
    hxt = hxt + lax.dot_general(wpw1t_ref[...], pos_ref[...],
                                (((1,), (1,)), ((), ())),
                                preferred_element_type=jnp.float32)
    hxt = hxt + b1_ref[...]
    ht = hxt * jax.nn.sigmoid(hxt)

    xat = jnp.dot(w2_ref[...], ht,
                  preferred_element_type=jnp.float32)
    xat = xat + b2_ref[0, 0]

    bb = batch_ref[0, :, :]
    seg = (jax.lax.broadcasted_iota(jnp.int32, (B, BLK), 0) == bb)
    masked = seg.astype(jnp.float32) * xat
    partial = jnp.dot(masked, jnp.ones((BLK, 1), jnp.float32),
                      preferred_element_type=jnp.float32)

    out_ref[...] += partial


@jax.jit
def kernel(z, pos, batch, emb, Wp, W1, b1, W2, b2):
    grid = N // BLK
    z3 = z.astype(jnp.int32).reshape(grid, 1, BLK)
    batch3 = batch.astype(jnp.int32).reshape(grid, 1, BLK)
    b1c = b1.reshape(H, 1)
    w2r = W2.reshape(1, H)
    b2r = b2.reshape(1, 1)

    out = pl.pallas_call(
        _fused_kernel,
        grid=(grid,),
        in_specs=[
            pl.BlockSpec((1, 1, BLK), lambda i: (i, 0, 0)),
            pl.BlockSpec((BLK, 3), lambda i: (i, 0)),
            pl.BlockSpec((1, 1, BLK), lambda i: (i, 0, 0)),
            pl.BlockSpec((ZMAXK, D), lambda i: (0, 0)),
            pl.BlockSpec((3, D), lambda i: (0, 0)),
            pl.BlockSpec((D, H), lambda i: (0, 0)),
            pl.BlockSpec((H, 1), lambda i: (0, 0)),
            pl.BlockSpec((1, H), lambda i: (0, 0)),
            pl.BlockSpec((1, 1), lambda i: (0, 0)),
        ],
        out_specs=pl.BlockSpec((B, 1), lambda i: (0, 0)),
        out_shape=jax.ShapeDtypeStruct((B, 1), jnp.float32),
        scratch_shapes=[
            pltpu.VMEM((H, ZPAD), jnp.float32),
            pltpu.VMEM((H, 3), jnp.float32),
        ],
    )(z3, pos, batch3, emb, Wp, W1, b1c, w2r, b2r)
    return out
